# baseline (speedup 1.0000x reference)
import functools

import jax
import jax.numpy as jnp
from jax import lax
from jax.experimental import pallas as pl
from jax.experimental.pallas import tpu as pltpu
from jax.experimental.pallas import tpu_sc as plsc

N_ROWS = 320000
D = 128
N_SEG = 10000
NW = 32
SEG_PER_W = 313
LAST_LO = N_SEG - SEG_PER_W
CHUNK = 256
GRP = 16
LANES = 8
NEG_INF = float("-inf")


def _tile_body(feats_r, batch_r, rlo_r, rhi_r, out_r,
               fb0, fb1, bb0, bb1, acc, lov, hiv, cfv, curs,
               sf0, sf1, sb0, sb1):
    wid = lax.axis_index("s") * 2 + lax.axis_index("c")
    pltpu.sync_copy(rlo_r, lov.at[pl.ds(0, NW)])
    pltpu.sync_copy(rhi_r, hiv.at[pl.ds(0, NW)])
    off0 = lov[pl.ds(wid, 16)][0]
    off1 = hiv[pl.ds(wid, 16)][0]
    lo_seg = jnp.minimum(wid * SEG_PER_W, LAST_LO)

    def init_body(i, _):
        acc[pl.ds(i * 16, 16)] = jnp.full((16,), NEG_INF, jnp.float32)
        return 0
    lax.fori_loop(0, SEG_PER_W * LANES, init_body, 0)
    for k in range(LANES):
        cfv[pl.ds(16 * k, 16)] = jnp.full((16,), NEG_INF, jnp.float32)
    curs[0] = jnp.int32(-1)

    c0 = off0 >> 8
    c1 = (off1 + (CHUNK - 1)) >> 8
    nc = c1 - c0
    nc2 = (nc + 1) & ~1

    def chunk_start(ci):
        return jnp.minimum((c0 + ci) * CHUNK, N_ROWS - CHUNK)

    def start_dma(ci, fb, bb, sf, sb):
        st = chunk_start(ci)
        pltpu.make_async_copy(
            feats_r.at[pl.ds(st * D, CHUNK * D)], fb, sf).start()
        pltpu.make_async_copy(
            batch_r.at[pl.ds(st, CHUNK)], bb.at[pl.ds(0, CHUNK)], sb).start()

    def load_a():
        return [cfv[pl.ds(16 * k, 16)] for k in range(LANES)]

    def save_a(a):
        for k in range(LANES):
            cfv[pl.ds(16 * k, 16)] = a[k]

    def flush(cur, a):
        base = jnp.clip(cur - lo_seg, 0, SEG_PER_W - 1) * D
        for k in range(LANES):
            acc[pl.ds(base + 16 * k, 16)] = a[k]

    def step(cur, a, s, rbase, fb):
        is_new = s != cur

        @pl.when(is_new)
        def _():
            flush(cur, a)

        pen = jnp.where(is_new, jnp.float32(NEG_INF), jnp.float32(0.0))
        penv = jnp.broadcast_to(pen, (16,))
        na = []
        for k in range(LANES):
            row_k = fb[pl.ds(rbase + 16 * k, 16)]
            na.append(jnp.maximum(a[k] + penv, row_k))
        return s, na

    def process(ci, fb, bb, sf, sb):
        pltpu.make_async_copy(
            feats_r.at[pl.ds(0, CHUNK * D)], fb, sf).wait()
        pltpu.make_async_copy(
            batch_r.at[pl.ds(0, CHUNK)], bb.at[pl.ds(0, CHUNK)], sb).wait()
        start = (c0 + ci) * CHUNK
        r_lo = jnp.maximum(off0 - start, 0)
        r_hi = jnp.minimum(off1 - start, CHUNK)

        def full_chunk():
            def group_body(gi, _):
                base_r = gi * GRP
                sv = bb[pl.ds(base_r, GRP)]
                cur = curs[0]
                cont = sv[GRP - 1] == cur

                @pl.when(cont)
                def _():
                    a = load_a()
                    for j in range(GRP):
                        rbase = (base_r + j) * D
                        for k in range(LANES):
                            a[k] = jnp.maximum(a[k],
                                               fb[pl.ds(rbase + 16 * k, 16)])
                    save_a(a)

                @pl.when(jnp.logical_not(cont))
                def _():
                    a = load_a()
                    c = cur
                    for j in range(GRP):
                        c, a = step(c, a, sv[j], (base_r + j) * D, fb)
                    save_a(a)
                    curs[0] = c

                return 0

            lax.fori_loop(0, CHUNK // GRP, group_body, 0)

        def ragged_chunk():
            def row_body(r, carry):
                c, na = step(carry[0], list(carry[1:]),
                             bb[pl.ds(r, 16)][0], r * D, fb)
                return (c,) + tuple(na)

            res = lax.fori_loop(r_lo, r_hi, row_body,
                                (curs[0],) + tuple(load_a()))
            save_a(list(res[1:]))
            curs[0] = res[0]

        is_full = jnp.logical_and(r_lo == 0, r_hi == CHUNK)

        @pl.when(is_full)
        def _():
            full_chunk()

        @pl.when(jnp.logical_not(is_full))
        def _():
            ragged_chunk()

    @pl.when(nc > 0)
    def _():
        start_dma(0, fb0, bb0, sf0, sb0)
        start_dma(1, fb1, bb1, sf1, sb1)

    def pair_body(h, _):
        g = 2 * h
        process(g, fb0, bb0, sf0, sb0)

        @pl.when(g + 2 < nc2)
        def _():
            start_dma(g + 2, fb0, bb0, sf0, sb0)

        process(g + 1, fb1, bb1, sf1, sb1)

        @pl.when(g + 3 < nc2)
        def _():
            start_dma(g + 3, fb1, bb1, sf1, sb1)

        return 0

    lax.fori_loop(0, nc2 >> 1, pair_body, 0)

    flush(curs[0], load_a())
    pltpu.sync_copy(acc, out_r.at[pl.ds(lo_seg * D, SEG_PER_W * D)])


@jax.jit
def _run(feats1d, batch, rlo, rhi):
    mesh = plsc.VectorSubcoreMesh(core_axis_name="c", subcore_axis_name="s")
    k = functools.partial(
        pl.kernel,
        mesh=mesh,
        out_type=jax.ShapeDtypeStruct((N_SEG * D,), jnp.float32),
        scratch_types=[
            pltpu.VMEM((CHUNK * D,), jnp.float32),
            pltpu.VMEM((CHUNK * D,), jnp.float32),
            pltpu.VMEM((CHUNK + 16,), jnp.int32),
            pltpu.VMEM((CHUNK + 16,), jnp.int32),
            pltpu.VMEM((SEG_PER_W * D,), jnp.float32),
            pltpu.VMEM((NW + 16,), jnp.int32),
            pltpu.VMEM((NW + 16,), jnp.int32),
            pltpu.VMEM((LANES * 16,), jnp.float32),
            pltpu.SMEM((8,), jnp.int32),
            pltpu.SemaphoreType.DMA,
            pltpu.SemaphoreType.DMA,
            pltpu.SemaphoreType.DMA,
            pltpu.SemaphoreType.DMA,
        ],
    )(_tile_body)
    return k(feats1d, batch, rlo, rhi)


def kernel(feats, batch):
    lo = jnp.minimum(jnp.arange(NW, dtype=jnp.int32) * SEG_PER_W, LAST_LO)
    thr = jnp.concatenate([lo, lo + SEG_PER_W])
    cnt = jnp.searchsorted(batch, thr, side="left",
                           method="compare_all").astype(jnp.int32)
    rlo, rhi = cnt[:NW], cnt[NW:]
    out = _run(feats.reshape(-1), batch, rlo, rhi)
    return out.reshape(N_SEG, D)


# speedup vs baseline: 1.0334x; 1.0334x over previous
import functools

import jax
import jax.numpy as jnp
from jax import lax
from jax.experimental import pallas as pl
from jax.experimental.pallas import tpu as pltpu
from jax.experimental.pallas import tpu_sc as plsc

N_ROWS = 320000
D = 128
N_SEG = 10000
NW = 32
SEG_PER_W = 313
LAST_LO = N_SEG - SEG_PER_W
CHUNK = 256
GRP = 16
LANES = 8
NEG_INF = float("-inf")


def _tile_body(feats_r, batch_r, rlo_r, rhi_r, out_r,
               fb0, fb1, bb0, bb1, acc, lov, hiv, cfv, curs,
               sf0, sf1, sb0, sb1):
    wid = lax.axis_index("s") * 2 + lax.axis_index("c")
    pltpu.sync_copy(rlo_r, lov.at[pl.ds(0, NW)])
    pltpu.sync_copy(rhi_r, hiv.at[pl.ds(0, NW)])
    off0 = lov[pl.ds(wid, 16)][0]
    off1 = hiv[pl.ds(wid, 16)][0]
    lo_seg = jnp.minimum(wid * SEG_PER_W, LAST_LO)

    def init_body(i, _):
        acc[pl.ds(i * 16, 16)] = jnp.full((16,), NEG_INF, jnp.float32)
        return 0
    lax.fori_loop(0, SEG_PER_W * LANES, init_body, 0)
    for k in range(LANES):
        cfv[pl.ds(16 * k, 16)] = jnp.full((16,), NEG_INF, jnp.float32)
    curs[0] = jnp.int32(-1)

    c0 = off0 >> 8
    c1 = (off1 + (CHUNK - 1)) >> 8
    nc = c1 - c0
    nc2 = (nc + 1) & ~1

    def chunk_start(ci):
        return jnp.minimum((c0 + ci) * CHUNK, N_ROWS - CHUNK)

    def start_dma(ci, fb, bb, sf, sb):
        st = chunk_start(ci)
        pltpu.make_async_copy(
            feats_r.at[pl.ds(st * D, CHUNK * D)], fb, sf).start()
        pltpu.make_async_copy(
            batch_r.at[pl.ds(st, CHUNK)], bb.at[pl.ds(0, CHUNK)], sb).start()

    def load_a():
        return [cfv[pl.ds(16 * k, 16)] for k in range(LANES)]

    def save_a(a):
        for k in range(LANES):
            cfv[pl.ds(16 * k, 16)] = a[k]

    def flush(cur, a):
        base = jnp.clip(cur - lo_seg, 0, SEG_PER_W - 1) * D
        for k in range(LANES):
            acc[pl.ds(base + 16 * k, 16)] = a[k]

    def step(cur, a, s, rbase, fb):
        is_new = s != cur
        base = jnp.where(is_new,
                         jnp.clip(cur - lo_seg, 0, SEG_PER_W - 1),
                         jnp.int32(SEG_PER_W)) * D
        pen = jnp.where(is_new, jnp.float32(NEG_INF), jnp.float32(0.0))
        penv = jnp.broadcast_to(pen, (16,))
        na = []
        for k in range(LANES):
            acc[pl.ds(base + 16 * k, 16)] = a[k]
            row_k = fb[pl.ds(rbase + 16 * k, 16)]
            na.append(jnp.maximum(a[k] + penv, row_k))
        return s, na

    def process(ci, fb, bb, sf, sb):
        pltpu.make_async_copy(
            feats_r.at[pl.ds(0, CHUNK * D)], fb, sf).wait()
        pltpu.make_async_copy(
            batch_r.at[pl.ds(0, CHUNK)], bb.at[pl.ds(0, CHUNK)], sb).wait()
        start = (c0 + ci) * CHUNK
        r_lo = jnp.maximum(off0 - start, 0)
        r_hi = jnp.minimum(off1 - start, CHUNK)

        def full_chunk():
            def group_body(gi, _):
                base_r = gi * GRP
                sv = bb[pl.ds(base_r, GRP)]
                cur = curs[0]
                cont = sv[GRP - 1] == cur

                @pl.when(cont)
                def _():
                    a = load_a()
                    for j in range(GRP):
                        rbase = (base_r + j) * D
                        for k in range(LANES):
                            a[k] = jnp.maximum(a[k],
                                               fb[pl.ds(rbase + 16 * k, 16)])
                    save_a(a)

                @pl.when(jnp.logical_not(cont))
                def _():
                    a = load_a()
                    c = cur
                    for j in range(GRP):
                        c, a = step(c, a, sv[j], (base_r + j) * D, fb)
                    save_a(a)
                    curs[0] = c

                return 0

            lax.fori_loop(0, CHUNK // GRP, group_body, 0)

        def ragged_chunk():
            def row_body(r, carry):
                c, na = step(carry[0], list(carry[1:]),
                             bb[pl.ds(r, 16)][0], r * D, fb)
                return (c,) + tuple(na)

            res = lax.fori_loop(r_lo, r_hi, row_body,
                                (curs[0],) + tuple(load_a()))
            save_a(list(res[1:]))
            curs[0] = res[0]

        is_full = jnp.logical_and(r_lo == 0, r_hi == CHUNK)

        @pl.when(is_full)
        def _():
            full_chunk()

        @pl.when(jnp.logical_not(is_full))
        def _():
            ragged_chunk()

    @pl.when(nc > 0)
    def _():
        start_dma(0, fb0, bb0, sf0, sb0)
        start_dma(1, fb1, bb1, sf1, sb1)

    def pair_body(h, _):
        g = 2 * h
        process(g, fb0, bb0, sf0, sb0)

        @pl.when(g + 2 < nc2)
        def _():
            start_dma(g + 2, fb0, bb0, sf0, sb0)

        process(g + 1, fb1, bb1, sf1, sb1)

        @pl.when(g + 3 < nc2)
        def _():
            start_dma(g + 3, fb1, bb1, sf1, sb1)

        return 0

    lax.fori_loop(0, nc2 >> 1, pair_body, 0)

    flush(curs[0], load_a())
    pltpu.sync_copy(acc.at[pl.ds(0, SEG_PER_W * D)],
                    out_r.at[pl.ds(lo_seg * D, SEG_PER_W * D)])


@jax.jit
def _run(feats1d, batch, rlo, rhi):
    mesh = plsc.VectorSubcoreMesh(core_axis_name="c", subcore_axis_name="s")
    k = functools.partial(
        pl.kernel,
        mesh=mesh,
        out_type=jax.ShapeDtypeStruct((N_SEG * D,), jnp.float32),
        scratch_types=[
            pltpu.VMEM((CHUNK * D,), jnp.float32),
            pltpu.VMEM((CHUNK * D,), jnp.float32),
            pltpu.VMEM((CHUNK + 16,), jnp.int32),
            pltpu.VMEM((CHUNK + 16,), jnp.int32),
            pltpu.VMEM(((SEG_PER_W + 1) * D,), jnp.float32),
            pltpu.VMEM((NW + 16,), jnp.int32),
            pltpu.VMEM((NW + 16,), jnp.int32),
            pltpu.VMEM((LANES * 16,), jnp.float32),
            pltpu.SMEM((8,), jnp.int32),
            pltpu.SemaphoreType.DMA,
            pltpu.SemaphoreType.DMA,
            pltpu.SemaphoreType.DMA,
            pltpu.SemaphoreType.DMA,
        ],
    )(_tile_body)
    return k(feats1d, batch, rlo, rhi)


def kernel(feats, batch):
    lo = jnp.minimum(jnp.arange(NW, dtype=jnp.int32) * SEG_PER_W, LAST_LO)
    thr = jnp.concatenate([lo, lo + SEG_PER_W])
    cnt = jnp.searchsorted(batch, thr, side="left",
                           method="compare_all").astype(jnp.int32)
    rlo, rhi = cnt[:NW], cnt[NW:]
    out = _run(feats.reshape(-1), batch, rlo, rhi)
    return out.reshape(N_SEG, D)
